# baseline (speedup 1.0000x reference)
import functools

import jax
import jax.numpy as jnp
from jax import lax
from jax.experimental import pallas as pl
from jax.experimental.pallas import tpu as pltpu
from jax.experimental.pallas import tpu_sc as plsc

N = 10000
E = 160000
K = 4
H = 8
CH = 32
D = H * CH
IN = 128
LH = 256

NP = 10240
NC = 2
NS = 16
H_PER = H // NC
ET = E // NS
CW = 80
NCH = ET // CW
QW = CW // 16
SLC = NP // NS
NRB = 4

BN = 2000
NBK = N // BN

_f32 = jnp.float32
_i32 = jnp.int32


def _tc1_body(x_ref, w_ref, as_ref, ad_ref, h_ref, als_ref, ald_ref):
    xb = x_ref[0]
    hb = jnp.dot(xb, w_ref[...], preferred_element_type=_f32)
    for c in range(NC):
        for p in range(2):
            h_ref[0, c, p] = hb[:, c * 128 + p * 64:c * 128 + (p + 1) * 64]
    als_ref[0, 0] = jnp.dot(hb, as_ref[...], preferred_element_type=_f32).T
    ald_ref[0, 0] = jnp.dot(hb, ad_ref[...], preferred_element_type=_f32).T


def _tc_layer1(x, W1, As, Ad):
    return pl.pallas_call(
        _tc1_body,
        grid=(K, NBK),
        in_specs=[
            pl.BlockSpec((1, BN, IN), lambda k, i: (k, i, 0)),
            pl.BlockSpec((IN, D), lambda k, i: (0, 0)),
            pl.BlockSpec((D, H), lambda k, i: (0, 0)),
            pl.BlockSpec((D, H), lambda k, i: (0, 0)),
        ],
        out_specs=[
            pl.BlockSpec((1, NC, 2, BN, 64), lambda k, i: (k, 0, 0, i, 0)),
            pl.BlockSpec((1, 1, H, BN), lambda k, i: (k, i, 0, 0)),
            pl.BlockSpec((1, 1, H, BN), lambda k, i: (k, i, 0, 0)),
        ],
        out_shape=[
            jax.ShapeDtypeStruct((K, NC, 2, N, 64), _f32),
            jax.ShapeDtypeStruct((K, NBK, H, BN), _f32),
            jax.ShapeDtypeStruct((K, NBK, H, BN), _f32),
        ],
    )(x, W1, As, Ad)


def _tc2_body(o_ref, b_ref, w_ref, as_ref, ad_ref, h_ref, als_ref, ald_ref):
    acc = jnp.zeros((BN, D), _f32)
    for hd in range(H):
        c, pj = divmod(hd, H_PER)
        p, j = divmod(pj, 2)
        xh = jnp.maximum(o_ref[0, c, p][:, j * CH:(j + 1) * CH] + b_ref[hd], 0.0)
        acc = acc + jnp.dot(xh, w_ref[hd * CH:(hd + 1) * CH, :],
                            preferred_element_type=_f32)
    for c in range(NC):
        for p in range(2):
            h_ref[0, c, p] = acc[:, c * 128 + p * 64:c * 128 + (p + 1) * 64]
    als_ref[0, 0] = jnp.dot(acc, as_ref[...], preferred_element_type=_f32).T
    ald_ref[0, 0] = jnp.dot(acc, ad_ref[...], preferred_element_type=_f32).T


def _tc_layer2(o1, b1h, W2, As, Ad):
    return pl.pallas_call(
        _tc2_body,
        grid=(K, NBK),
        in_specs=[
            pl.BlockSpec((1, NC, 2, BN, 64), lambda k, i: (k, 0, 0, i, 0)),
            pl.BlockSpec((H, CH), lambda k, i: (0, 0)),
            pl.BlockSpec((D, D), lambda k, i: (0, 0)),
            pl.BlockSpec((D, H), lambda k, i: (0, 0)),
            pl.BlockSpec((D, H), lambda k, i: (0, 0)),
        ],
        out_specs=[
            pl.BlockSpec((1, NC, 2, BN, 64), lambda k, i: (k, 0, 0, i, 0)),
            pl.BlockSpec((1, 1, H, BN), lambda k, i: (k, i, 0, 0)),
            pl.BlockSpec((1, 1, H, BN), lambda k, i: (k, i, 0, 0)),
        ],
        out_shape=[
            jax.ShapeDtypeStruct((K, NC, 2, N, 64), _f32),
            jax.ShapeDtypeStruct((K, NBK, H, BN), _f32),
            jax.ShapeDtypeStruct((K, NBK, H, BN), _f32),
        ],
    )(o1, b1h, W2, As, Ad)


def _sc_alpha_body(als, ald, esrc, edst, alpha,
                   src_v, dst_v, tbl_s, tbl_d, ex4, zv, den_sh):
    c = lax.axis_index("c")
    s = lax.axis_index("s")
    base = s * ET
    z16 = jnp.zeros((16,), _f32)

    def _zb(i, carry):
        zv[pl.ds(i * 16, 16)] = z16
        return carry
    lax.fori_loop(0, SLC // 16, _zb, 0)

    def _per_k(k, carry):
        pltpu.sync_copy(esrc.at[pl.ds(k * E + base, ET)], src_v)
        pltpu.sync_copy(edst.at[pl.ds(k * E + base, ET)], dst_v)

        def _zden(j, carry2):
            pltpu.sync_copy(zv, den_sh.at[j, pl.ds(s * SLC, SLC)])
            return carry2
        lax.fori_loop(0, H_PER, _zden, 0)
        plsc.subcore_barrier()

        def _per_j(j, carry2):
            hd = c * H_PER + j

            def _tl(ib, carry3):
                off = ((k * NBK + ib) * H + hd) * BN
                pltpu.sync_copy(als.at[pl.ds(off, BN)],
                                tbl_s.at[pl.ds(ib * BN, BN)])
                pltpu.sync_copy(ald.at[pl.ds(off, BN)],
                                tbl_d.at[pl.ds(ib * BN, BN)])
                return carry3
            lax.fori_loop(0, NBK, _tl, 0)

            def _edges(g, carry3):
                sv = src_v[pl.ds(g * 16, 16)]
                dv = dst_v[pl.ds(g * 16, 16)]
                a = plsc.load_gather(tbl_s, [sv])
                b = plsc.load_gather(tbl_d, [dv])
                e = a + b
                e = jnp.where(e >= 0.0, e, e * _f32(0.2))
                ex4[j, pl.ds(g * 16, 16)] = jnp.exp(e)
                return carry3
            lax.fori_loop(0, ET // 16, _edges, 0)

            pltpu.sync_copy(ex4.at[j], den_sh.at[j].at[dst_v], add=True)
            return carry2
        lax.fori_loop(0, H_PER, _per_j, 0)
        plsc.subcore_barrier()

        def _per_j2(j, carry2):
            hd = c * H_PER + j
            pltpu.sync_copy(den_sh.at[j], tbl_d)

            def _norm(g, carry3):
                dv = dst_v[pl.ds(g * 16, 16)]
                dn = plsc.load_gather(tbl_d, [dv])
                ex4[j, pl.ds(g * 16, 16)] = (
                    ex4[j, pl.ds(g * 16, 16)] / (dn + _f32(1e-16)))
                return carry3
            lax.fori_loop(0, ET // 16, _norm, 0)

            pltpu.sync_copy(ex4.at[j],
                            alpha.at[pl.ds((k * H + hd) * E + base, ET)])
            return carry2
        lax.fori_loop(0, H_PER, _per_j2, 0)
        plsc.subcore_barrier()
        return carry
    lax.fori_loop(0, K, _per_k, 0)


def _sc_alpha(als, ald, esrc, edst):
    mesh = plsc.VectorSubcoreMesh(core_axis_name="c", subcore_axis_name="s")
    kfn = functools.partial(
        pl.kernel,
        out_type=jax.ShapeDtypeStruct((K * H * E,), _f32),
        mesh=mesh,
        scratch_types=[
            pltpu.VMEM((ET,), _i32),
            pltpu.VMEM((ET,), _i32),
            pltpu.VMEM((N,), _f32),
            pltpu.VMEM((NP,), _f32),
            pltpu.VMEM((H_PER, ET), _f32),
            pltpu.VMEM((SLC,), _f32),
            pltpu.VMEM_SHARED((H_PER, NP), _f32),
        ],
        compiler_params=pltpu.CompilerParams(needs_layout_passes=False, use_tc_tiling_on_sc=False),
    )(_sc_alpha_body)
    return kfn(als, ald, esrc, edst)


def _sc_agg_body(h4, alpha, esrc, edst, out,
                 src_v, dst_v, dst2, alpha2, rows, zr, out_sh, gsem, ssem):
    c = lax.axis_index("c")
    s = lax.axis_index("s")
    base = s * ET
    z16 = jnp.zeros((16,), _f32)

    def _zb(i, carry):
        for t in range(4):
            zr[i, pl.ds(t * 16, 16)] = z16
        return carry
    lax.fori_loop(0, 16, _zb, 0)

    def _scale(ci, pb):
        def _sq(q, carry):
            a0 = alpha2[0, pl.ds(ci * CW + q * 16, 16)]
            a1 = alpha2[1, pl.ds(ci * CW + q * 16, 16)]
            heads = (a0, a1)
            for l in range(16):
                sel = jnp.full((16,), l, _i32)
                e = q * 16 + l
                for j in range(2):
                    bc = heads[j].at[sel].get(mode="promise_in_bounds")
                    r0 = rows[pb, e, pl.ds(j * CH, 16)] * bc
                    rows[pb, e, pl.ds(j * CH, 16)] = r0
                    r1 = rows[pb, e, pl.ds(j * CH + 16, 16)] * bc
                    rows[pb, e, pl.ds(j * CH + 16, 16)] = r1
            return carry
        lax.fori_loop(0, QW, _sq, 0)

    def _issue_gather(k, c, p, ci, pb):
        return pltpu.async_copy(
            h4.at[k, c, p].at[src_v.at[pl.ds(ci * CW, CW)]],
            rows.at[pb], gsem.at[pb])

    def _wait_gather(k, c, p, ci, pb):
        pltpu.make_async_copy(
            h4.at[k, c, p].at[src_v.at[pl.ds(ci * CW, CW)]],
            rows.at[pb], gsem.at[pb]).wait()

    def _issue_scatter(ci, pb):
        pltpu.async_copy(rows.at[pb], out_sh.at[dst2.at[ci]],
                         ssem.at[pb], add=True)

    def _wait_scatter(ci, pb):
        pltpu.make_async_copy(rows.at[pb], out_sh.at[dst2.at[ci]],
                              ssem.at[pb]).wait()

    def _per_k(k, carry):
        pltpu.sync_copy(esrc.at[pl.ds(k * E + base, ET)], src_v)
        pltpu.sync_copy(edst.at[pl.ds(k * E + base, ET)], dst_v)

        def _cpy(ci, carry2):
            for q in range(QW):
                dst2[ci, pl.ds(q * 16, 16)] = dst_v[pl.ds(ci * CW + q * 16,
                                                          16)]
            return carry2
        lax.fori_loop(0, NCH, _cpy, 0)

        def _per_p(p, carry2):
            def _al(j, carry3):
                hd = c * H_PER + p * 2 + j
                pltpu.sync_copy(alpha.at[pl.ds((k * H + hd) * E + base, ET)],
                                alpha2.at[j])
                return carry3
            lax.fori_loop(0, 2, _al, 0)

            def _zo(i, carry3):
                pltpu.sync_copy(zr, out_sh.at[pl.ds(s * SLC + i * 16, 16)])
                return carry3
            lax.fori_loop(0, SLC // 16, _zo, 0)
            plsc.subcore_barrier()

            def _chunk(ci, carry3):
                pb = lax.rem(ci, NRB)

                @pl.when(ci >= NRB)
                def _():
                    _wait_scatter(ci - NRB, pb)
                _issue_gather(k, c, p, ci, pb)

                @pl.when(ci >= 2)
                def _():
                    pb2 = lax.rem(ci - 2, NRB)
                    _wait_gather(k, c, p, ci - 2, pb2)
                    _scale(ci - 2, pb2)
                    _issue_scatter(ci - 2, pb2)
                return carry3
            lax.fori_loop(0, NCH, _chunk, 0)
            for cj in (NCH - 2, NCH - 1):
                pb = cj % NRB
                _wait_gather(k, c, p, cj, pb)
                _scale(cj, pb)
                _issue_scatter(cj, pb)
            for cj in range(NCH - NRB, NCH):
                _wait_scatter(cj, cj % NRB)

            plsc.subcore_barrier()
            pltpu.sync_copy(out_sh.at[pl.ds(s * SLC, SLC)],
                            out.at[k, c, p, pl.ds(s * SLC, SLC)])
            plsc.subcore_barrier()
            return carry2
        lax.fori_loop(0, 2, _per_p, 0)
        return carry
    lax.fori_loop(0, K, _per_k, 0)


def _sc_aggregate(h4, alpha, esrc, edst):
    mesh = plsc.VectorSubcoreMesh(core_axis_name="c", subcore_axis_name="s")
    kfn = functools.partial(
        pl.kernel,
        out_type=jax.ShapeDtypeStruct((K, NC, 2, NP, 64), _f32),
        mesh=mesh,
        scratch_types=[
            pltpu.VMEM((ET,), _i32),
            pltpu.VMEM((ET,), _i32),
            pltpu.VMEM((NCH, CW), _i32),
            pltpu.VMEM((2, ET), _f32),
            pltpu.VMEM((NRB, CW, 64), _f32),
            pltpu.VMEM((16, 64), _f32),
            pltpu.VMEM_SHARED((NP, 64), _f32),
            pltpu.SemaphoreType.DMA((NRB,)),
            pltpu.SemaphoreType.DMA((NRB,)),
        ],
        compiler_params=pltpu.CompilerParams(needs_layout_passes=False, use_tc_tiling_on_sc=False),
    )(_sc_agg_body)
    return kfn(h4, alpha, esrc, edst)


def _tc3_body(o_ref, b_ref, wif_ref, whf_ref, bf_ref, wib_ref, whb_ref,
              bb_ref, out_ref, acc_ref):
    i = pl.program_id(0)

    @pl.when(i == 0)
    def _():
        acc_ref[...] = jnp.zeros((K, NC, 2, 64), _f32)

    blk = jnp.maximum(o_ref[...] + b_ref[...][None, :, :, None, :], 0.0)
    acc_ref[...] = acc_ref[...] + jnp.sum(blk, axis=3)

    @pl.when(i == NBK - 1)
    def _():
        seq = acc_ref[...].reshape(K, D) * _f32(1.0 / N)

        def run(order, wih, whh, bsum):
            hf = jnp.zeros((1, LH), _f32)
            cf = jnp.zeros((1, LH), _f32)
            for t in order:
                g = (jnp.dot(seq[t:t + 1], wih, preferred_element_type=_f32)
                     + jnp.dot(hf, whh, preferred_element_type=_f32)
                     + bsum[None, :])
                ig = jax.nn.sigmoid(g[:, :LH])
                fg = jax.nn.sigmoid(g[:, LH:2 * LH])
                gg = jnp.tanh(g[:, 2 * LH:3 * LH])
                og = jax.nn.sigmoid(g[:, 3 * LH:])
                cf = fg * cf + ig * gg
                hf = og * jnp.tanh(cf)
            return hf

        hfwd = run(range(K), wif_ref[...], whf_ref[...], bf_ref[...])
        hbwd = run(range(K - 1, -1, -1), wib_ref[...], whb_ref[...],
                   bb_ref[...])
        out_ref[...] = jnp.concatenate([hfwd, hbwd], axis=-1)


def _tc_pool_lstm(o2, b2c, WihT_f, WhhT_f, bs_f, WihT_b, WhhT_b, bs_b):
    return pl.pallas_call(
        _tc3_body,
        grid=(NBK,),
        in_specs=[
            pl.BlockSpec((K, NC, 2, BN, 64), lambda i: (0, 0, 0, i, 0)),
            pl.BlockSpec((NC, 2, 64), lambda i: (0, 0, 0)),
            pl.BlockSpec((D, 4 * LH), lambda i: (0, 0)),
            pl.BlockSpec((LH, 4 * LH), lambda i: (0, 0)),
            pl.BlockSpec((4 * LH,), lambda i: (0,)),
            pl.BlockSpec((D, 4 * LH), lambda i: (0, 0)),
            pl.BlockSpec((LH, 4 * LH), lambda i: (0, 0)),
            pl.BlockSpec((4 * LH,), lambda i: (0,)),
        ],
        out_specs=pl.BlockSpec((1, 2 * LH), lambda i: (0, 0)),
        out_shape=jax.ShapeDtypeStruct((1, 2 * LH), _f32),
        scratch_shapes=[pltpu.VMEM((K, NC, 2, 64), _f32)],
    )(o2, b2c, WihT_f, WhhT_f, bs_f, WihT_b, WhhT_b, bs_b)


def _head_mat(a):
    a = a.reshape(H, CH)
    return (a[:, :, None] * jnp.eye(H, dtype=a.dtype)[:, None, :]).reshape(D, H)


def kernel(x, edge_index, W1, a_src1, a_dst1, b1, W2, a_src2, a_dst2, b2,
           Wih_f, Whh_f, bih_f, bhh_f, Wih_b, Whh_b, bih_b, bhh_b):
    eidx = edge_index.astype(_i32)
    esrc = eidx[:, 0, :].reshape(K * E)
    edst = eidx[:, 1, :].reshape(K * E)
    As1, Ad1 = _head_mat(a_src1), _head_mat(a_dst1)
    As2, Ad2 = _head_mat(a_src2), _head_mat(a_dst2)
    b1h = b1.reshape(H, CH)
    b2c = b2.reshape(NC, 2, 64)

    h1, als1, ald1 = _tc_layer1(x, W1, As1, Ad1)
    alpha1 = _sc_alpha(als1.reshape(-1), ald1.reshape(-1), esrc, edst)
    o1 = _sc_aggregate(h1, alpha1, esrc, edst)

    h2, als2, ald2 = _tc_layer2(o1, b1h, W2, As2, Ad2)
    alpha2 = _sc_alpha(als2.reshape(-1), ald2.reshape(-1), esrc, edst)
    o2 = _sc_aggregate(h2, alpha2, esrc, edst)

    return _tc_pool_lstm(o2, b2c, Wih_f.T, Whh_f.T, bih_f + bhh_f,
                         Wih_b.T, Whh_b.T, bih_b + bhh_b)


# speedup vs baseline: 1.3520x; 1.3520x over previous
import functools

import jax
import jax.numpy as jnp
from jax import lax
from jax.experimental import pallas as pl
from jax.experimental.pallas import tpu as pltpu
from jax.experimental.pallas import tpu_sc as plsc

N = 10000
E = 160000
K = 4
H = 8
CH = 32
D = H * CH
IN = 128
LH = 256

NP = 10240
NC = 2
NS = 16
H_PER = H // NC
ET = E // NS
CW = 80
NCH = ET // CW
QW = CW // 16
SLC = NP // NS
NRB = 4

BN = 2000
NBK = N // BN

_f32 = jnp.float32
_i32 = jnp.int32


def _tc1_body(x_ref, w_ref, as_ref, ad_ref, h_ref, als_ref, ald_ref):
    xb = x_ref[0]
    hb = jnp.dot(xb, w_ref[...], preferred_element_type=_f32)
    for c in range(NC):
        for p in range(2):
            h_ref[0, c, p] = hb[:, c * 128 + p * 64:c * 128 + (p + 1) * 64]
    als_ref[0, 0] = jnp.dot(hb, as_ref[...], preferred_element_type=_f32).T
    ald_ref[0, 0] = jnp.dot(hb, ad_ref[...], preferred_element_type=_f32).T


def _tc_layer1(x, W1, As, Ad):
    return pl.pallas_call(
        _tc1_body,
        grid=(K, NBK),
        in_specs=[
            pl.BlockSpec((1, BN, IN), lambda k, i: (k, i, 0)),
            pl.BlockSpec((IN, D), lambda k, i: (0, 0)),
            pl.BlockSpec((D, H), lambda k, i: (0, 0)),
            pl.BlockSpec((D, H), lambda k, i: (0, 0)),
        ],
        out_specs=[
            pl.BlockSpec((1, NC, 2, BN, 64), lambda k, i: (k, 0, 0, i, 0)),
            pl.BlockSpec((1, 1, H, BN), lambda k, i: (k, i, 0, 0)),
            pl.BlockSpec((1, 1, H, BN), lambda k, i: (k, i, 0, 0)),
        ],
        out_shape=[
            jax.ShapeDtypeStruct((K, NC, 2, N, 64), _f32),
            jax.ShapeDtypeStruct((K, NBK, H, BN), _f32),
            jax.ShapeDtypeStruct((K, NBK, H, BN), _f32),
        ],
    )(x, W1, As, Ad)


def _tc2_body(o_ref, b_ref, w_ref, as_ref, ad_ref, h_ref, als_ref, ald_ref):
    acc = jnp.zeros((BN, D), _f32)
    for hd in range(H):
        c, pj = divmod(hd, H_PER)
        p, j = divmod(pj, 2)
        xh = jnp.maximum(o_ref[0, c, p][:, j * CH:(j + 1) * CH] + b_ref[hd], 0.0)
        acc = acc + jnp.dot(xh, w_ref[hd * CH:(hd + 1) * CH, :],
                            preferred_element_type=_f32)
    for c in range(NC):
        for p in range(2):
            h_ref[0, c, p] = acc[:, c * 128 + p * 64:c * 128 + (p + 1) * 64]
    als_ref[0, 0] = jnp.dot(acc, as_ref[...], preferred_element_type=_f32).T
    ald_ref[0, 0] = jnp.dot(acc, ad_ref[...], preferred_element_type=_f32).T


def _tc_layer2(o1, b1h, W2, As, Ad):
    return pl.pallas_call(
        _tc2_body,
        grid=(K, NBK),
        in_specs=[
            pl.BlockSpec((1, NC, 2, BN, 64), lambda k, i: (k, 0, 0, i, 0)),
            pl.BlockSpec((H, CH), lambda k, i: (0, 0)),
            pl.BlockSpec((D, D), lambda k, i: (0, 0)),
            pl.BlockSpec((D, H), lambda k, i: (0, 0)),
            pl.BlockSpec((D, H), lambda k, i: (0, 0)),
        ],
        out_specs=[
            pl.BlockSpec((1, NC, 2, BN, 64), lambda k, i: (k, 0, 0, i, 0)),
            pl.BlockSpec((1, 1, H, BN), lambda k, i: (k, i, 0, 0)),
            pl.BlockSpec((1, 1, H, BN), lambda k, i: (k, i, 0, 0)),
        ],
        out_shape=[
            jax.ShapeDtypeStruct((K, NC, 2, N, 64), _f32),
            jax.ShapeDtypeStruct((K, NBK, H, BN), _f32),
            jax.ShapeDtypeStruct((K, NBK, H, BN), _f32),
        ],
    )(o1, b1h, W2, As, Ad)


def _sc_alpha_body(als, ald, esrc, edst, alpha,
                   src_v, dst_v, tbl_s, tbl_d, ex4, zv, den_sh):
    c = lax.axis_index("c")
    s = lax.axis_index("s")
    base = s * ET
    z16 = jnp.zeros((16,), _f32)

    def _zb(i, carry):
        zv[pl.ds(i * 16, 16)] = z16
        return carry
    lax.fori_loop(0, SLC // 16, _zb, 0)

    def _per_k(k, carry):
        pltpu.sync_copy(esrc.at[pl.ds(k * E + base, ET)], src_v)
        pltpu.sync_copy(edst.at[pl.ds(k * E + base, ET)], dst_v)

        def _zden(j, carry2):
            pltpu.sync_copy(zv, den_sh.at[j, pl.ds(s * SLC, SLC)])
            return carry2
        lax.fori_loop(0, H_PER, _zden, 0)
        plsc.subcore_barrier()

        def _per_j(j, carry2):
            hd = c * H_PER + j

            def _tl(ib, carry3):
                off = ((k * NBK + ib) * H + hd) * BN
                pltpu.sync_copy(als.at[pl.ds(off, BN)],
                                tbl_s.at[pl.ds(ib * BN, BN)])
                pltpu.sync_copy(ald.at[pl.ds(off, BN)],
                                tbl_d.at[pl.ds(ib * BN, BN)])
                return carry3
            lax.fori_loop(0, NBK, _tl, 0)

            @plsc.parallel_loop(0, ET // 16, unroll=4)
            def _edges(g):
                sv = src_v[pl.ds(g * 16, 16)]
                dv = dst_v[pl.ds(g * 16, 16)]
                a = plsc.load_gather(tbl_s, [sv])
                b = plsc.load_gather(tbl_d, [dv])
                e = a + b
                e = jnp.where(e >= 0.0, e, e * _f32(0.2))
                ex4[j, pl.ds(g * 16, 16)] = jnp.exp(e)

            pltpu.sync_copy(ex4.at[j], den_sh.at[j].at[dst_v], add=True)
            return carry2
        lax.fori_loop(0, H_PER, _per_j, 0)
        plsc.subcore_barrier()

        def _per_j2(j, carry2):
            hd = c * H_PER + j
            pltpu.sync_copy(den_sh.at[j], tbl_d)

            @plsc.parallel_loop(0, ET // 16, unroll=4)
            def _norm(g):
                dv = dst_v[pl.ds(g * 16, 16)]
                dn = plsc.load_gather(tbl_d, [dv])
                ex4[j, pl.ds(g * 16, 16)] = (
                    ex4[j, pl.ds(g * 16, 16)] / (dn + _f32(1e-16)))

            pltpu.sync_copy(ex4.at[j],
                            alpha.at[pl.ds((k * H + hd) * E + base, ET)])
            return carry2
        lax.fori_loop(0, H_PER, _per_j2, 0)
        plsc.subcore_barrier()
        return carry
    lax.fori_loop(0, K, _per_k, 0)


def _sc_alpha(als, ald, esrc, edst):
    mesh = plsc.VectorSubcoreMesh(core_axis_name="c", subcore_axis_name="s")
    kfn = functools.partial(
        pl.kernel,
        out_type=jax.ShapeDtypeStruct((K * H * E,), _f32),
        mesh=mesh,
        scratch_types=[
            pltpu.VMEM((ET,), _i32),
            pltpu.VMEM((ET,), _i32),
            pltpu.VMEM((N,), _f32),
            pltpu.VMEM((NP,), _f32),
            pltpu.VMEM((H_PER, ET), _f32),
            pltpu.VMEM((SLC,), _f32),
            pltpu.VMEM_SHARED((H_PER, NP), _f32),
        ],
        compiler_params=pltpu.CompilerParams(needs_layout_passes=False, use_tc_tiling_on_sc=False),
    )(_sc_alpha_body)
    return kfn(als, ald, esrc, edst)


def _sc_agg_body(h4, alpha, esrc, edst, out,
                 src_v, dst_v, dst2, alpha2, rows, zr, out_sh, gsem, ssem):
    c = lax.axis_index("c")
    s = lax.axis_index("s")
    base = s * ET
    z16 = jnp.zeros((16,), _f32)

    def _zb(i, carry):
        for t in range(4):
            zr[i, pl.ds(t * 16, 16)] = z16
        return carry
    lax.fori_loop(0, 16, _zb, 0)

    def _scale(ci, pb):
        @plsc.parallel_loop(0, QW)
        def _sq(q):
            a0 = alpha2[0, pl.ds(ci * CW + q * 16, 16)]
            a1 = alpha2[1, pl.ds(ci * CW + q * 16, 16)]
            heads = (a0, a1)
            for l in range(16):
                sel = jnp.full((16,), l, _i32)
                e = q * 16 + l
                for j in range(2):
                    bc = heads[j].at[sel].get(mode="promise_in_bounds")
                    r0 = rows[pb, e, pl.ds(j * CH, 16)] * bc
                    rows[pb, e, pl.ds(j * CH, 16)] = r0
                    r1 = rows[pb, e, pl.ds(j * CH + 16, 16)] * bc
                    rows[pb, e, pl.ds(j * CH + 16, 16)] = r1

    def _issue_gather(k, c, p, ci, pb):
        return pltpu.async_copy(
            h4.at[k, c, p].at[src_v.at[pl.ds(ci * CW, CW)]],
            rows.at[pb], gsem.at[pb])

    def _wait_gather(k, c, p, ci, pb):
        pltpu.make_async_copy(
            h4.at[k, c, p].at[src_v.at[pl.ds(ci * CW, CW)]],
            rows.at[pb], gsem.at[pb]).wait()

    def _issue_scatter(ci, pb):
        pltpu.async_copy(rows.at[pb], out_sh.at[dst2.at[ci]],
                         ssem.at[pb], add=True)

    def _wait_scatter(ci, pb):
        pltpu.make_async_copy(rows.at[pb], out_sh.at[dst2.at[ci]],
                              ssem.at[pb]).wait()

    def _per_k(k, carry):
        pltpu.sync_copy(esrc.at[pl.ds(k * E + base, ET)], src_v)
        pltpu.sync_copy(edst.at[pl.ds(k * E + base, ET)], dst_v)

        @plsc.parallel_loop(0, NCH, unroll=2)
        def _cpy(ci):
            for q in range(QW):
                dst2[ci, pl.ds(q * 16, 16)] = dst_v[pl.ds(ci * CW + q * 16,
                                                          16)]

        def _per_p(p, carry2):
            def _al(j, carry3):
                hd = c * H_PER + p * 2 + j
                pltpu.sync_copy(alpha.at[pl.ds((k * H + hd) * E + base, ET)],
                                alpha2.at[j])
                return carry3
            lax.fori_loop(0, 2, _al, 0)

            def _zo(i, carry3):
                pltpu.sync_copy(zr, out_sh.at[pl.ds(s * SLC + i * 16, 16)])
                return carry3
            lax.fori_loop(0, SLC // 16, _zo, 0)
            plsc.subcore_barrier()

            def _chunk(ci, carry3):
                pb = lax.rem(ci, NRB)

                @pl.when(ci >= NRB)
                def _():
                    _wait_scatter(ci - NRB, pb)
                _issue_gather(k, c, p, ci, pb)

                @pl.when(ci >= 2)
                def _():
                    pb2 = lax.rem(ci - 2, NRB)
                    _wait_gather(k, c, p, ci - 2, pb2)
                    _scale(ci - 2, pb2)
                    _issue_scatter(ci - 2, pb2)
                return carry3
            lax.fori_loop(0, NCH, _chunk, 0)
            for cj in (NCH - 2, NCH - 1):
                pb = cj % NRB
                _wait_gather(k, c, p, cj, pb)
                _scale(cj, pb)
                _issue_scatter(cj, pb)
            for cj in range(NCH - NRB, NCH):
                _wait_scatter(cj, cj % NRB)

            plsc.subcore_barrier()
            pltpu.sync_copy(out_sh.at[pl.ds(s * SLC, SLC)],
                            out.at[k, c, p, pl.ds(s * SLC, SLC)])
            plsc.subcore_barrier()
            return carry2
        lax.fori_loop(0, 2, _per_p, 0)
        return carry
    lax.fori_loop(0, K, _per_k, 0)


def _sc_aggregate(h4, alpha, esrc, edst):
    mesh = plsc.VectorSubcoreMesh(core_axis_name="c", subcore_axis_name="s")
    kfn = functools.partial(
        pl.kernel,
        out_type=jax.ShapeDtypeStruct((K, NC, 2, NP, 64), _f32),
        mesh=mesh,
        scratch_types=[
            pltpu.VMEM((ET,), _i32),
            pltpu.VMEM((ET,), _i32),
            pltpu.VMEM((NCH, CW), _i32),
            pltpu.VMEM((2, ET), _f32),
            pltpu.VMEM((NRB, CW, 64), _f32),
            pltpu.VMEM((16, 64), _f32),
            pltpu.VMEM_SHARED((NP, 64), _f32),
            pltpu.SemaphoreType.DMA((NRB,)),
            pltpu.SemaphoreType.DMA((NRB,)),
        ],
        compiler_params=pltpu.CompilerParams(needs_layout_passes=False, use_tc_tiling_on_sc=False),
    )(_sc_agg_body)
    return kfn(h4, alpha, esrc, edst)


def _tc3_body(o_ref, b_ref, wif_ref, whf_ref, bf_ref, wib_ref, whb_ref,
              bb_ref, out_ref, acc_ref):
    i = pl.program_id(0)

    @pl.when(i == 0)
    def _():
        acc_ref[...] = jnp.zeros((K, NC, 2, 64), _f32)

    blk = jnp.maximum(o_ref[...] + b_ref[...][None, :, :, None, :], 0.0)
    acc_ref[...] = acc_ref[...] + jnp.sum(blk, axis=3)

    @pl.when(i == NBK - 1)
    def _():
        seq = acc_ref[...].reshape(K, D) * _f32(1.0 / N)

        def run(order, wih, whh, bsum):
            hf = jnp.zeros((1, LH), _f32)
            cf = jnp.zeros((1, LH), _f32)
            for t in order:
                g = (jnp.dot(seq[t:t + 1], wih, preferred_element_type=_f32)
                     + jnp.dot(hf, whh, preferred_element_type=_f32)
                     + bsum[None, :])
                ig = jax.nn.sigmoid(g[:, :LH])
                fg = jax.nn.sigmoid(g[:, LH:2 * LH])
                gg = jnp.tanh(g[:, 2 * LH:3 * LH])
                og = jax.nn.sigmoid(g[:, 3 * LH:])
                cf = fg * cf + ig * gg
                hf = og * jnp.tanh(cf)
            return hf

        hfwd = run(range(K), wif_ref[...], whf_ref[...], bf_ref[...])
        hbwd = run(range(K - 1, -1, -1), wib_ref[...], whb_ref[...],
                   bb_ref[...])
        out_ref[...] = jnp.concatenate([hfwd, hbwd], axis=-1)


def _tc_pool_lstm(o2, b2c, WihT_f, WhhT_f, bs_f, WihT_b, WhhT_b, bs_b):
    return pl.pallas_call(
        _tc3_body,
        grid=(NBK,),
        in_specs=[
            pl.BlockSpec((K, NC, 2, BN, 64), lambda i: (0, 0, 0, i, 0)),
            pl.BlockSpec((NC, 2, 64), lambda i: (0, 0, 0)),
            pl.BlockSpec((D, 4 * LH), lambda i: (0, 0)),
            pl.BlockSpec((LH, 4 * LH), lambda i: (0, 0)),
            pl.BlockSpec((4 * LH,), lambda i: (0,)),
            pl.BlockSpec((D, 4 * LH), lambda i: (0, 0)),
            pl.BlockSpec((LH, 4 * LH), lambda i: (0, 0)),
            pl.BlockSpec((4 * LH,), lambda i: (0,)),
        ],
        out_specs=pl.BlockSpec((1, 2 * LH), lambda i: (0, 0)),
        out_shape=jax.ShapeDtypeStruct((1, 2 * LH), _f32),
        scratch_shapes=[pltpu.VMEM((K, NC, 2, 64), _f32)],
    )(o2, b2c, WihT_f, WhhT_f, bs_f, WihT_b, WhhT_b, bs_b)


def _head_mat(a):
    a = a.reshape(H, CH)
    return (a[:, :, None] * jnp.eye(H, dtype=a.dtype)[:, None, :]).reshape(D, H)


def kernel(x, edge_index, W1, a_src1, a_dst1, b1, W2, a_src2, a_dst2, b2,
           Wih_f, Whh_f, bih_f, bhh_f, Wih_b, Whh_b, bih_b, bhh_b):
    eidx = edge_index.astype(_i32)
    esrc = eidx[:, 0, :].reshape(K * E)
    edst = eidx[:, 1, :].reshape(K * E)
    As1, Ad1 = _head_mat(a_src1), _head_mat(a_dst1)
    As2, Ad2 = _head_mat(a_src2), _head_mat(a_dst2)
    b1h = b1.reshape(H, CH)
    b2c = b2.reshape(NC, 2, 64)

    h1, als1, ald1 = _tc_layer1(x, W1, As1, Ad1)
    alpha1 = _sc_alpha(als1.reshape(-1), ald1.reshape(-1), esrc, edst)
    o1 = _sc_aggregate(h1, alpha1, esrc, edst)

    h2, als2, ald2 = _tc_layer2(o1, b1h, W2, As2, Ad2)
    alpha2 = _sc_alpha(als2.reshape(-1), ald2.reshape(-1), esrc, edst)
    o2 = _sc_aggregate(h2, alpha2, esrc, edst)

    return _tc_pool_lstm(o2, b2c, Wih_f.T, Whh_f.T, bih_f + bhh_f,
                         Wih_b.T, Whh_b.T, bih_b + bhh_b)


# speedup vs baseline: 1.5741x; 1.1643x over previous
import functools

import jax
import jax.numpy as jnp
from jax import lax
from jax.experimental import pallas as pl
from jax.experimental.pallas import tpu as pltpu
from jax.experimental.pallas import tpu_sc as plsc

N = 10000
E = 160000
K = 4
H = 8
CH = 32
D = H * CH
IN = 128
LH = 256

NP = 10240
NC = 2
NS = 16
H_PER = H // NC
ET = E // NS
CW = 80
NCH = ET // CW
QW = CW // 16
SLC = NP // NS
NRB = 4
ZR = 80

BN = 2000
NBK = N // BN

_f32 = jnp.float32
_i32 = jnp.int32


def _tc1_body(x_ref, w_ref, as_ref, ad_ref, h_ref, als_ref, ald_ref):
    xb = x_ref[0]
    hb = jnp.dot(xb, w_ref[...], preferred_element_type=_f32)
    for c in range(NC):
        for p in range(2):
            h_ref[0, c, p] = hb[:, c * 128 + p * 64:c * 128 + (p + 1) * 64]
    als_ref[0, 0] = jnp.dot(hb, as_ref[...], preferred_element_type=_f32).T
    ald_ref[0, 0] = jnp.dot(hb, ad_ref[...], preferred_element_type=_f32).T


def _tc_layer1(x, W1, As, Ad):
    return pl.pallas_call(
        _tc1_body,
        grid=(K, NBK),
        in_specs=[
            pl.BlockSpec((1, BN, IN), lambda k, i: (k, i, 0)),
            pl.BlockSpec((IN, D), lambda k, i: (0, 0)),
            pl.BlockSpec((D, H), lambda k, i: (0, 0)),
            pl.BlockSpec((D, H), lambda k, i: (0, 0)),
        ],
        out_specs=[
            pl.BlockSpec((1, NC, 2, BN, 64), lambda k, i: (k, 0, 0, i, 0)),
            pl.BlockSpec((1, 1, H, BN), lambda k, i: (k, i, 0, 0)),
            pl.BlockSpec((1, 1, H, BN), lambda k, i: (k, i, 0, 0)),
        ],
        out_shape=[
            jax.ShapeDtypeStruct((K, NC, 2, N, 64), _f32),
            jax.ShapeDtypeStruct((K, NBK, H, BN), _f32),
            jax.ShapeDtypeStruct((K, NBK, H, BN), _f32),
        ],
    )(x, W1, As, Ad)


def _tc2_body(o_ref, b_ref, w_ref, as_ref, ad_ref, h_ref, als_ref, ald_ref):
    acc = jnp.zeros((BN, D), _f32)
    for hd in range(H):
        c, pj = divmod(hd, H_PER)
        p, j = divmod(pj, 2)
        xh = jnp.maximum(o_ref[0, c, p][:, j * CH:(j + 1) * CH] + b_ref[hd], 0.0)
        acc = acc + jnp.dot(xh, w_ref[hd * CH:(hd + 1) * CH, :],
                            preferred_element_type=_f32)
    for c in range(NC):
        for p in range(2):
            h_ref[0, c, p] = acc[:, c * 128 + p * 64:c * 128 + (p + 1) * 64]
    als_ref[0, 0] = jnp.dot(acc, as_ref[...], preferred_element_type=_f32).T
    ald_ref[0, 0] = jnp.dot(acc, ad_ref[...], preferred_element_type=_f32).T


def _tc_layer2(o1, b1h, W2, As, Ad):
    return pl.pallas_call(
        _tc2_body,
        grid=(K, NBK),
        in_specs=[
            pl.BlockSpec((1, NC, 2, BN, 64), lambda k, i: (k, 0, 0, i, 0)),
            pl.BlockSpec((H, CH), lambda k, i: (0, 0)),
            pl.BlockSpec((D, D), lambda k, i: (0, 0)),
            pl.BlockSpec((D, H), lambda k, i: (0, 0)),
            pl.BlockSpec((D, H), lambda k, i: (0, 0)),
        ],
        out_specs=[
            pl.BlockSpec((1, NC, 2, BN, 64), lambda k, i: (k, 0, 0, i, 0)),
            pl.BlockSpec((1, 1, H, BN), lambda k, i: (k, i, 0, 0)),
            pl.BlockSpec((1, 1, H, BN), lambda k, i: (k, i, 0, 0)),
        ],
        out_shape=[
            jax.ShapeDtypeStruct((K, NC, 2, N, 64), _f32),
            jax.ShapeDtypeStruct((K, NBK, H, BN), _f32),
            jax.ShapeDtypeStruct((K, NBK, H, BN), _f32),
        ],
    )(o1, b1h, W2, As, Ad)


def _sc_alpha_body(als, ald, esrc, edst, alpha,
                   src_v, dst_v, tbl_s, tbl_d, ex4, zv, den_sh, tsem, ssem):
    c = lax.axis_index("c")
    s = lax.axis_index("s")
    base = s * ET
    z16 = jnp.zeros((16,), _f32)

    def _zb(i, carry):
        zv[pl.ds(i * 16, 16)] = z16
        return carry
    lax.fori_loop(0, SLC // 16, _zb, 0)

    def _per_k(k, carry):
        pltpu.sync_copy(esrc.at[pl.ds(k * E + base, ET)], src_v)
        pltpu.sync_copy(edst.at[pl.ds(k * E + base, ET)], dst_v)

        def _zden(j, carry2):
            pltpu.sync_copy(zv, den_sh.at[j, pl.ds(s * SLC, SLC)])
            return carry2
        lax.fori_loop(0, H_PER, _zden, 0)
        plsc.subcore_barrier()

        def _per_j(j, carry2):
            hd = c * H_PER + j

            descs = []
            for ib in range(NBK):
                off = ((k * NBK + ib) * H + hd) * BN
                descs.append(pltpu.async_copy(
                    als.at[pl.ds(off, BN)], tbl_s.at[pl.ds(ib * BN, BN)],
                    tsem))
                descs.append(pltpu.async_copy(
                    ald.at[pl.ds(off, BN)], tbl_d.at[pl.ds(ib * BN, BN)],
                    tsem))
            for d in descs:
                d.wait()

            @plsc.parallel_loop(0, ET // 16, unroll=4)
            def _edges(g):
                sv = src_v[pl.ds(g * 16, 16)]
                dv = dst_v[pl.ds(g * 16, 16)]
                a = plsc.load_gather(tbl_s, [sv])
                b = plsc.load_gather(tbl_d, [dv])
                e = a + b
                e = jnp.where(e >= 0.0, e, e * _f32(0.2))
                ex4[j, pl.ds(g * 16, 16)] = jnp.exp(e)

            pltpu.async_copy(ex4.at[j], den_sh.at[j].at[dst_v], ssem,
                             add=True)
            return carry2
        lax.fori_loop(0, H_PER, _per_j, 0)

        def _dr(j, carry2):
            pltpu.make_async_copy(ex4.at[j], den_sh.at[j].at[dst_v],
                                  ssem).wait()
            return carry2
        lax.fori_loop(0, H_PER, _dr, 0)
        plsc.subcore_barrier()

        def _per_j2(j, carry2):
            hd = c * H_PER + j
            pltpu.sync_copy(den_sh.at[j], tbl_d)

            @plsc.parallel_loop(0, ET // 16, unroll=4)
            def _norm(g):
                dv = dst_v[pl.ds(g * 16, 16)]
                dn = plsc.load_gather(tbl_d, [dv])
                ex4[j, pl.ds(g * 16, 16)] = (
                    ex4[j, pl.ds(g * 16, 16)] / (dn + _f32(1e-16)))

            pltpu.sync_copy(ex4.at[j],
                            alpha.at[pl.ds((k * H + hd) * E + base, ET)])
            return carry2
        lax.fori_loop(0, H_PER, _per_j2, 0)
        plsc.subcore_barrier()
        return carry
    lax.fori_loop(0, K, _per_k, 0)


def _sc_alpha(als, ald, esrc, edst):
    mesh = plsc.VectorSubcoreMesh(core_axis_name="c", subcore_axis_name="s")
    kfn = functools.partial(
        pl.kernel,
        out_type=jax.ShapeDtypeStruct((K * H * E,), _f32),
        mesh=mesh,
        scratch_types=[
            pltpu.VMEM((ET,), _i32),
            pltpu.VMEM((ET,), _i32),
            pltpu.VMEM((N,), _f32),
            pltpu.VMEM((NP,), _f32),
            pltpu.VMEM((H_PER, ET), _f32),
            pltpu.VMEM((SLC,), _f32),
            pltpu.VMEM_SHARED((H_PER, NP), _f32),
            pltpu.SemaphoreType.DMA,
            pltpu.SemaphoreType.DMA,
        ],
        compiler_params=pltpu.CompilerParams(needs_layout_passes=False, use_tc_tiling_on_sc=False),
    )(_sc_alpha_body)
    return kfn(als, ald, esrc, edst)


def _sc_agg_body(h4, alpha, esrc, edst, out,
                 src_v, dst_v, dst2, alpha2, rows, zr, out_sh, gsem, ssem,
                 zsem):
    c = lax.axis_index("c")
    s = lax.axis_index("s")
    base = s * ET
    z16 = jnp.zeros((16,), _f32)

    def _zb(i, carry):
        for t in range(4):
            zr[i, pl.ds(t * 16, 16)] = z16
        return carry
    lax.fori_loop(0, ZR, _zb, 0)

    def _scale(ci, pb):
        @plsc.parallel_loop(0, QW)
        def _sq(q):
            a0 = alpha2[0, pl.ds(ci * CW + q * 16, 16)]
            a1 = alpha2[1, pl.ds(ci * CW + q * 16, 16)]
            heads = (a0, a1)
            for l in range(16):
                sel = jnp.full((16,), l, _i32)
                e = q * 16 + l
                for j in range(2):
                    bc = heads[j].at[sel].get(mode="promise_in_bounds")
                    r0 = rows[pb, e, pl.ds(j * CH, 16)] * bc
                    rows[pb, e, pl.ds(j * CH, 16)] = r0
                    r1 = rows[pb, e, pl.ds(j * CH + 16, 16)] * bc
                    rows[pb, e, pl.ds(j * CH + 16, 16)] = r1

    def _issue_gather(k, c, p, ci, pb):
        return pltpu.async_copy(
            h4.at[k, c, p].at[src_v.at[pl.ds(ci * CW, CW)]],
            rows.at[pb], gsem.at[pb])

    def _wait_gather(k, c, p, ci, pb):
        pltpu.make_async_copy(
            h4.at[k, c, p].at[src_v.at[pl.ds(ci * CW, CW)]],
            rows.at[pb], gsem.at[pb]).wait()

    def _issue_scatter(ci, pb):
        pltpu.async_copy(rows.at[pb], out_sh.at[dst2.at[ci]],
                         ssem.at[pb], add=True)

    def _wait_scatter(ci, pb):
        pltpu.make_async_copy(rows.at[pb], out_sh.at[dst2.at[ci]],
                              ssem.at[pb]).wait()

    def _per_k(k, carry):
        pltpu.sync_copy(esrc.at[pl.ds(k * E + base, ET)], src_v)
        pltpu.sync_copy(edst.at[pl.ds(k * E + base, ET)], dst_v)

        @plsc.parallel_loop(0, NCH, unroll=2)
        def _cpy(ci):
            for q in range(QW):
                dst2[ci, pl.ds(q * 16, 16)] = dst_v[pl.ds(ci * CW + q * 16,
                                                          16)]

        def _per_p(p, carry2):
            def _al(j, carry3):
                hd = c * H_PER + p * 2 + j
                pltpu.sync_copy(alpha.at[pl.ds((k * H + hd) * E + base, ET)],
                                alpha2.at[j])
                return carry3
            lax.fori_loop(0, 2, _al, 0)

            zdescs = [pltpu.async_copy(
                zr, out_sh.at[pl.ds(s * SLC + i * ZR, ZR)], zsem)
                for i in range(SLC // ZR)]
            for zd in zdescs:
                zd.wait()
            plsc.subcore_barrier()

            def _chunk(ci, carry3):
                pb = lax.rem(ci, NRB)

                @pl.when(ci >= NRB)
                def _():
                    _wait_scatter(ci - NRB, pb)
                _issue_gather(k, c, p, ci, pb)

                @pl.when(ci >= 2)
                def _():
                    pb2 = lax.rem(ci - 2, NRB)
                    _wait_gather(k, c, p, ci - 2, pb2)
                    _scale(ci - 2, pb2)
                    _issue_scatter(ci - 2, pb2)
                return carry3
            lax.fori_loop(0, NCH, _chunk, 0)
            for cj in (NCH - 2, NCH - 1):
                pb = cj % NRB
                _wait_gather(k, c, p, cj, pb)
                _scale(cj, pb)
                _issue_scatter(cj, pb)
            for cj in range(NCH - NRB, NCH):
                _wait_scatter(cj, cj % NRB)

            plsc.subcore_barrier()
            pltpu.sync_copy(out_sh.at[pl.ds(s * SLC, SLC)],
                            out.at[k, c, p, pl.ds(s * SLC, SLC)])
            plsc.subcore_barrier()
            return carry2
        lax.fori_loop(0, 2, _per_p, 0)
        return carry
    lax.fori_loop(0, K, _per_k, 0)


def _sc_aggregate(h4, alpha, esrc, edst):
    mesh = plsc.VectorSubcoreMesh(core_axis_name="c", subcore_axis_name="s")
    kfn = functools.partial(
        pl.kernel,
        out_type=jax.ShapeDtypeStruct((K, NC, 2, NP, 64), _f32),
        mesh=mesh,
        scratch_types=[
            pltpu.VMEM((ET,), _i32),
            pltpu.VMEM((ET,), _i32),
            pltpu.VMEM((NCH, CW), _i32),
            pltpu.VMEM((2, ET), _f32),
            pltpu.VMEM((NRB, CW, 64), _f32),
            pltpu.VMEM((ZR, 64), _f32),
            pltpu.VMEM_SHARED((NP, 64), _f32),
            pltpu.SemaphoreType.DMA((NRB,)),
            pltpu.SemaphoreType.DMA((NRB,)),
            pltpu.SemaphoreType.DMA,
        ],
        compiler_params=pltpu.CompilerParams(needs_layout_passes=False, use_tc_tiling_on_sc=False),
    )(_sc_agg_body)
    return kfn(h4, alpha, esrc, edst)


def _tc3_body(o_ref, b_ref, wif_ref, whf_ref, bf_ref, wib_ref, whb_ref,
              bb_ref, out_ref, acc_ref):
    i = pl.program_id(0)

    @pl.when(i == 0)
    def _():
        acc_ref[...] = jnp.zeros((K, NC, 2, 64), _f32)

    blk = jnp.maximum(o_ref[...] + b_ref[...][None, :, :, None, :], 0.0)
    acc_ref[...] = acc_ref[...] + jnp.sum(blk, axis=3)

    @pl.when(i == NBK - 1)
    def _():
        seq = acc_ref[...].reshape(K, D) * _f32(1.0 / N)

        def run(order, wih, whh, bsum):
            hf = jnp.zeros((1, LH), _f32)
            cf = jnp.zeros((1, LH), _f32)
            for t in order:
                g = (jnp.dot(seq[t:t + 1], wih, preferred_element_type=_f32)
                     + jnp.dot(hf, whh, preferred_element_type=_f32)
                     + bsum[None, :])
                ig = jax.nn.sigmoid(g[:, :LH])
                fg = jax.nn.sigmoid(g[:, LH:2 * LH])
                gg = jnp.tanh(g[:, 2 * LH:3 * LH])
                og = jax.nn.sigmoid(g[:, 3 * LH:])
                cf = fg * cf + ig * gg
                hf = og * jnp.tanh(cf)
            return hf

        hfwd = run(range(K), wif_ref[...], whf_ref[...], bf_ref[...])
        hbwd = run(range(K - 1, -1, -1), wib_ref[...], whb_ref[...],
                   bb_ref[...])
        out_ref[...] = jnp.concatenate([hfwd, hbwd], axis=-1)


def _tc_pool_lstm(o2, b2c, WihT_f, WhhT_f, bs_f, WihT_b, WhhT_b, bs_b):
    return pl.pallas_call(
        _tc3_body,
        grid=(NBK,),
        in_specs=[
            pl.BlockSpec((K, NC, 2, BN, 64), lambda i: (0, 0, 0, i, 0)),
            pl.BlockSpec((NC, 2, 64), lambda i: (0, 0, 0)),
            pl.BlockSpec((D, 4 * LH), lambda i: (0, 0)),
            pl.BlockSpec((LH, 4 * LH), lambda i: (0, 0)),
            pl.BlockSpec((4 * LH,), lambda i: (0,)),
            pl.BlockSpec((D, 4 * LH), lambda i: (0, 0)),
            pl.BlockSpec((LH, 4 * LH), lambda i: (0, 0)),
            pl.BlockSpec((4 * LH,), lambda i: (0,)),
        ],
        out_specs=pl.BlockSpec((1, 2 * LH), lambda i: (0, 0)),
        out_shape=jax.ShapeDtypeStruct((1, 2 * LH), _f32),
        scratch_shapes=[pltpu.VMEM((K, NC, 2, 64), _f32)],
    )(o2, b2c, WihT_f, WhhT_f, bs_f, WihT_b, WhhT_b, bs_b)


def _head_mat(a):
    a = a.reshape(H, CH)
    return (a[:, :, None] * jnp.eye(H, dtype=a.dtype)[:, None, :]).reshape(D, H)


def kernel(x, edge_index, W1, a_src1, a_dst1, b1, W2, a_src2, a_dst2, b2,
           Wih_f, Whh_f, bih_f, bhh_f, Wih_b, Whh_b, bih_b, bhh_b):
    eidx = edge_index.astype(_i32)
    esrc = eidx[:, 0, :].reshape(K * E)
    edst = eidx[:, 1, :].reshape(K * E)
    As1, Ad1 = _head_mat(a_src1), _head_mat(a_dst1)
    As2, Ad2 = _head_mat(a_src2), _head_mat(a_dst2)
    b1h = b1.reshape(H, CH)
    b2c = b2.reshape(NC, 2, 64)

    h1, als1, ald1 = _tc_layer1(x, W1, As1, Ad1)
    alpha1 = _sc_alpha(als1.reshape(-1), ald1.reshape(-1), esrc, edst)
    o1 = _sc_aggregate(h1, alpha1, esrc, edst)

    h2, als2, ald2 = _tc_layer2(o1, b1h, W2, As2, Ad2)
    alpha2 = _sc_alpha(als2.reshape(-1), ald2.reshape(-1), esrc, edst)
    o2 = _sc_aggregate(h2, alpha2, esrc, edst)

    return _tc_pool_lstm(o2, b2c, Wih_f.T, Whh_f.T, bih_f + bhh_f,
                         Wih_b.T, Whh_b.T, bih_b + bhh_b)
